# Initial kernel scaffold; baseline (speedup 1.0000x reference)
import jax
import jax.numpy as jnp
from jax.experimental import pallas as pl


def kernel(x_spatial, global_data, edge_index_spatial, edge_attr_spatial, bip_edge_index, bip_edge_attr, x_species, x_species_phylo, edge_index_species, edge_attr_species, params):
    raise NotImplementedError("write your pallas kernel here")


# speedup vs baseline: 1.9524x; 1.9524x over previous
import functools

import jax
import jax.numpy as jnp
import numpy as np
from jax.experimental import pallas as pl
from jax.experimental.pallas import tpu as pltpu

HID = 64
AUGW = 80


def _mm_body(x_ref, w_ref, o_ref):
    o_ref[...] = jnp.dot(x_ref[...], w_ref[...],
                         preferred_element_type=jnp.float32)


def _matmul(x, w, bm=400):
    n, k = x.shape
    _, m = w.shape
    return pl.pallas_call(
        _mm_body,
        grid=(n // bm,),
        in_specs=[pl.BlockSpec((bm, k), lambda i: (i, 0)),
                  pl.BlockSpec((k, m), lambda i: (0, 0))],
        out_specs=pl.BlockSpec((bm, m), lambda i: (i, 0)),
        out_shape=jax.ShapeDtypeStruct((n, m), jnp.float32),
    )(x, w)


def _augment_weights(p):
    W = p['W']
    din = W.shape[0]
    cols = [W, (W @ p['att_src'])[:, None], (W @ p['att_dst'])[:, None],
            jnp.zeros((din, AUGW - HID - 2), jnp.float32)]
    return jnp.concatenate(cols, axis=1)


def _edge_pass_xla(H, src, dst, ea, num_dst, use_dst):
    asrc = H[:, HID]
    adst = H[:, HID + 1]
    h = H[:, :HID]
    alpha = asrc[src] + ea
    if use_dst:
        alpha = alpha + adst[dst]
    alpha = jnp.where(alpha >= 0, alpha, 0.2 * alpha)
    ex = jnp.exp(alpha)
    num = jax.ops.segment_sum(ex[:, None] * h[src], dst, num_segments=num_dst)
    exsum = jax.ops.segment_sum(ex, dst, num_segments=num_dst)
    easum = jax.ops.segment_sum(ea, dst, num_segments=num_dst)
    cnt = jax.ops.segment_sum(jnp.ones_like(ea), dst, num_segments=num_dst)
    pad = jnp.zeros((num_dst, AUGW - HID - 3), jnp.float32)
    return jnp.concatenate(
        [num, exsum[:, None], easum[:, None], cnt[:, None], pad], axis=1)


def _gat_layer(x, p, src, dst, edge_attr, num_dst, self_loops, use_dst):
    c = jnp.dot(p['W_e'][0], p['att_edge'])
    ea = c * edge_attr[:, 0]
    Waug = _augment_weights(p)
    H = _matmul(x, Waug)
    acc = _edge_pass_xla(H, src, dst, ea, num_dst, use_dst)
    num, exsum, easum, cnt = (acc[:, :HID], acc[:, HID],
                              acc[:, HID + 1], acc[:, HID + 2])
    if self_loops:
        h = H[:, :HID]
        asrc = H[:, HID]
        adst = H[:, HID + 1]
        loop_ea = easum / jnp.maximum(cnt, 1.0)
        a_loop = asrc + adst + loop_ea
        a_loop = jnp.where(a_loop >= 0, a_loop, 0.2 * a_loop)
        ex_loop = jnp.exp(a_loop)
        num = num + ex_loop[:, None] * h
        denom = exsum + ex_loop + 1e-16
    else:
        denom = exsum + 1e-16
    return num / denom[:, None] + p['b']


def kernel(x_spatial, global_data, edge_index_spatial, edge_attr_spatial,
           bip_edge_index, bip_edge_attr, x_species, x_species_phylo,
           edge_index_species, edge_attr_species, params):
    n_space = x_spatial.shape[0]
    n_species = x_species.shape[0]
    s_src, s_dst = edge_index_spatial[0], edge_index_spatial[1]
    b_src, b_dst = bip_edge_index[0], bip_edge_index[1]
    p_src, p_dst = edge_index_species[0], edge_index_species[1]

    si = jnp.concatenate([x_spatial, global_data], axis=1)
    h = jax.nn.relu(_gat_layer(si, params['space0'], s_src, s_dst,
                               edge_attr_spatial, n_space, True, True))
    h = _gat_layer(h, params['space1'], s_src, s_dst,
                   edge_attr_spatial, n_space, True, True)
    h = jax.nn.relu(h)
    s2s = _gat_layer(h, params['bip'], b_src, b_dst,
                     bip_edge_attr, n_species, False, False)
    spin = jnp.concatenate([s2s, x_species, x_species_phylo], axis=1)
    g = jax.nn.relu(_gat_layer(spin, params['sp0'], p_src, p_dst,
                               edge_attr_species, n_species, True, True))
    g = _gat_layer(g, params['sp1'], p_src, p_dst,
                   edge_attr_species, n_species, True, True)
    g = jax.nn.relu(g)
    return _matmul(g, params['fc_W']) + params['fc_b']


# speedup vs baseline: 3.3967x; 1.7398x over previous
import functools

import jax
import jax.numpy as jnp
import numpy as np
from jax import lax
from jax.experimental import pallas as pl
from jax.experimental.pallas import tpu as pltpu
from jax.experimental.pallas import tpu_sc as plsc

HID = 64
AUGW = 80
AUXW = 16
LANES = 16
EC = 256


def _mm_body(x_ref, w_ref, o_ref):
    o_ref[...] = jnp.dot(x_ref[...], w_ref[...],
                         preferred_element_type=jnp.float32)


def _matmul(x, w, bm=400):
    n, k = x.shape
    _, m = w.shape
    return pl.pallas_call(
        _mm_body,
        grid=(n // bm,),
        in_specs=[pl.BlockSpec((bm, k), lambda i: (i, 0)),
                  pl.BlockSpec((k, m), lambda i: (0, 0))],
        out_specs=pl.BlockSpec((bm, m), lambda i: (i, 0)),
        out_shape=jax.ShapeDtypeStruct((n, m), jnp.float32),
    )(x, w)


def _augment_weights(p):
    W = p['W']
    din = W.shape[0]
    cols = [W, (W @ p['att_src'])[:, None], (W @ p['att_dst'])[:, None],
            jnp.zeros((din, AUGW - HID - 2), jnp.float32)]
    return jnp.concatenate(cols, axis=1)


@functools.lru_cache(maxsize=None)
def _make_edge_kernel(e_pad, nloc, zc, split, use_dst, step):
    nrows = nloc // 16
    nz = nrows // zc
    assert nrows % zc == 0 and zc <= EC and nloc % 16 == 0
    per = e_pad // 16 if split else e_pad // 32
    nch = per // EC
    assert per % EC == 0

    mesh = plsc.VectorSubcoreMesh(core_axis_name="c", subcore_axis_name="s")

    @functools.partial(
        pl.kernel, mesh=mesh,
        compiler_params=pltpu.CompilerParams(needs_layout_passes=False,
                                             use_tc_tiling_on_sc=False),
        out_type=(jax.ShapeDtypeStruct((2, nloc, HID), jnp.float32),
                  jax.ShapeDtypeStruct((2, nloc, AUXW), jnp.float32)),
        scratch_types=[
            pltpu.VMEM((EC,), jnp.int32),
            pltpu.VMEM((EC,), jnp.int32),
            pltpu.VMEM((EC,), jnp.float32),
            pltpu.VMEM((EC,), jnp.int32),
            pltpu.VMEM((EC,), jnp.int32),
            pltpu.VMEM((EC,), jnp.float32),
            pltpu.VMEM((EC,), jnp.float32),
            pltpu.VMEM((EC, HID), jnp.float32),
            pltpu.VMEM((EC, AUXW), jnp.float32),
            pltpu.VMEM_SHARED((nloc, HID), jnp.float32),
            pltpu.VMEM_SHARED((nloc, AUXW), jnp.float32),
            pltpu.SemaphoreType.DMA,
        ],
    )
    def ek(h64, srcp, dstp, eap, asrcp, adstp, out64, outA,
           src_v, dst_v, ea_v, dstl_v, dstg_v, asrc_c, adst_c,
           rows_v, aux_v, acc64, accA, sem):
        c = lax.axis_index("c")
        s = lax.axis_index("s")
        ar = lax.iota(jnp.int32, LANES)
        zf = jnp.zeros((LANES,), jnp.float32)
        is0, is1, is2 = (ar == 0), (ar == 1), (ar == 2)

        def _zr(r, carry):
            for k in range(HID // LANES):
                rows_v[r, pl.ds(k * LANES, LANES)] = zf
            aux_v[r, pl.ds(0, LANES)] = zf
            return carry
        lax.fori_loop(0, zc, _zr, 0)
        r0 = s * nrows
        for z in range(nz):
            pltpu.sync_copy(rows_v.at[pl.ds(0, zc)],
                            acc64.at[pl.ds(r0 + z * zc, zc)])
            pltpu.sync_copy(aux_v.at[pl.ds(0, zc)],
                            accA.at[pl.ds(r0 + z * zc, zc)])
        plsc.subcore_barrier()

        dst_lo = (2 * step + c) * nloc if split else 0
        base = s * per if split else (s * 2 + c) * per

        def _chunk(g, carry):
            eb = base + g * EC
            pltpu.sync_copy(srcp.at[pl.ds(eb, EC)], src_v)
            pltpu.sync_copy(dstp.at[pl.ds(eb, EC)], dst_v)
            pltpu.sync_copy(eap.at[pl.ds(eb, EC)], ea_v)
            pltpu.async_copy(h64.at[src_v], rows_v, sem).wait()
            pltpu.async_copy(asrcp.at[src_v], asrc_c, sem).wait()

            def _grpA(j, carry2):
                jo = j * LANES
                dst16 = dst_v[pl.ds(jo, LANES)]
                dstl = dst16 - dst_lo
                m = (dstl >= 0) & (dstl < nloc)
                dstl_v[pl.ds(jo, LANES)] = jnp.where(m, dstl, 0)
                dstg_v[pl.ds(jo, LANES)] = jnp.where(m, dst16, 0)
                return carry2
            lax.fori_loop(0, EC // LANES, _grpA, 0)
            if use_dst:
                pltpu.async_copy(adstp.at[dstg_v], adst_c, sem).wait()

            def _grpB(j, carry2):
                jo = j * LANES
                dstl = dstl_v[pl.ds(jo, LANES)]
                dst16 = dst_v[pl.ds(jo, LANES)]
                ea16 = ea_v[pl.ds(jo, LANES)]
                m = (dst16 - dst_lo >= 0) & (dst16 - dst_lo < nloc)
                a = asrc_c[pl.ds(jo, LANES)] + ea16
                if use_dst:
                    a = a + adst_c[pl.ds(jo, LANES)]
                a = jnp.where(a >= 0.0, a, 0.2 * a)
                ex = jnp.where(m, jnp.exp(a), 0.0)
                one = jnp.where(m, 1.0, 0.0)
                eam = jnp.where(m, ea16, 0.0)
                for jj in range(LANES):
                    sel = ar - ar + jj
                    bex = ex.at[sel].get(mode='promise_in_bounds')
                    bea = eam.at[sel].get(mode='promise_in_bounds')
                    bone = one.at[sel].get(mode='promise_in_bounds')
                    aux = jnp.where(is0, bex,
                                    jnp.where(is1, bea,
                                              jnp.where(is2, bone, zf)))
                    aux_v[jo + jj, pl.ds(0, LANES)] = aux
                    for k in range(HID // LANES):
                        sl = pl.ds(k * LANES, LANES)
                        rows_v[jo + jj, sl] = rows_v[jo + jj, sl] * bex
                return carry2
            lax.fori_loop(0, EC // LANES, _grpB, 0)
            pltpu.sync_copy(rows_v, acc64.at[dstl_v], add=True)
            pltpu.sync_copy(aux_v, accA.at[dstl_v], add=True)
            return carry
        lax.fori_loop(0, nch, _chunk, 0)
        plsc.subcore_barrier()
        pltpu.sync_copy(acc64.at[pl.ds(r0, nrows)],
                        out64.at[c, pl.ds(r0, nrows)])
        pltpu.sync_copy(accA.at[pl.ds(r0, nrows)],
                        outA.at[c, pl.ds(r0, nrows)])

    return ek


def _pad1(a, n, fill):
    return jnp.pad(a, (0, n - a.shape[0]), constant_values=fill)


def _edge_pass_sc(h64, src_p, dst_p, ea_p, asrc_arr, adst_arr, num_dst,
                  nloc, zc, split, use_dst):
    n_steps = 2 if split else 1
    p64, pA = [], []
    for step in range(n_steps):
        ek = _make_edge_kernel(src_p.shape[0], nloc, zc, split, use_dst,
                               step)
        o64, oA = ek(h64, src_p, dst_p, ea_p, asrc_arr, adst_arr)
        p64 += [o64[0], o64[1]]
        pA += [oA[0], oA[1]]
    if split:
        return (jnp.concatenate(p64, axis=0)[:num_dst],
                jnp.concatenate(pA, axis=0)[:num_dst])
    return (p64[0] + p64[1])[:num_dst], (pA[0] + pA[1])[:num_dst]


def _gat_layer(x, p, src_p, dst_p, ea_raw_p, num_dst, self_loops, use_dst,
               nloc, zc, split):
    c = jnp.dot(p['W_e'][0], p['att_edge'])
    ea_p = c * ea_raw_p
    Waug = _augment_weights(p)
    H = _matmul(x, Waug)
    h64 = H[:, :HID]
    asrc_arr = H[:, HID]
    adst_arr = H[:, HID + 1] if use_dst else jnp.zeros((16,), jnp.float32)
    num, stats = _edge_pass_sc(h64, src_p, dst_p, ea_p, asrc_arr, adst_arr,
                               num_dst, nloc, zc, split, use_dst)
    exsum, easum, cnt = stats[:, 0], stats[:, 1], stats[:, 2]
    if self_loops:
        loop_ea = easum / jnp.maximum(cnt, 1.0)
        a_loop = asrc_arr + adst_arr + loop_ea
        a_loop = jnp.where(a_loop >= 0, a_loop, 0.2 * a_loop)
        ex_loop = jnp.exp(a_loop)
        num = num + ex_loop[:, None] * h64
        denom = exsum + ex_loop + 1e-16
    else:
        denom = exsum + 1e-16
    return num / denom[:, None] + p['b']


def kernel(x_spatial, global_data, edge_index_spatial, edge_attr_spatial,
           bip_edge_index, bip_edge_attr, x_species, x_species_phylo,
           edge_index_species, edge_attr_species, params):
    n_space = x_spatial.shape[0]
    n_species = x_species.shape[0]

    def _prep(src, dst, ea, e_pad):
        return (_pad1(src, e_pad, 0), _pad1(dst, e_pad, -1),
                _pad1(ea[:, 0], e_pad, 0.0))

    sp_e = _prep(edge_index_spatial[0], edge_index_spatial[1],
                 edge_attr_spatial, 802816)
    bp_e = _prep(bip_edge_index[0], bip_edge_index[1],
                 bip_edge_attr, 507904)
    pc_e = _prep(edge_index_species[0], edge_index_species[1],
                 edge_attr_species, 163840)

    SPG = dict(nloc=12544, zc=196, split=True)
    SCG = dict(nloc=10000, zc=125, split=False)

    si = jnp.concatenate([x_spatial, global_data], axis=1)
    h = jax.nn.relu(_gat_layer(si, params['space0'], *sp_e, n_space,
                               True, True, **SPG))
    h = _gat_layer(h, params['space1'], *sp_e, n_space, True, True, **SPG)
    h = jax.nn.relu(h)
    s2s = _gat_layer(h, params['bip'], *bp_e, n_species, False, False, **SCG)
    spin = jnp.concatenate([s2s, x_species, x_species_phylo], axis=1)
    g = jax.nn.relu(_gat_layer(spin, params['sp0'], *pc_e, n_species,
                               True, True, **SCG))
    g = _gat_layer(g, params['sp1'], *pc_e, n_species, True, True, **SCG)
    g = jax.nn.relu(g)
    return _matmul(g, params['fc_W']) + params['fc_b']


# speedup vs baseline: 9.7215x; 2.8620x over previous
import functools

import jax
import jax.numpy as jnp
import numpy as np
from jax import lax
from jax.experimental import pallas as pl
from jax.experimental.pallas import tpu as pltpu
from jax.experimental.pallas import tpu_sc as plsc

HID = 64
AUGW = 80
AUXW = 16
LANES = 16
EC = 256


def _mm_body(x_ref, w_ref, o_ref):
    o_ref[...] = jnp.dot(x_ref[...], w_ref[...],
                         preferred_element_type=jnp.float32)


def _matmul(x, w, bm=400):
    n, k = x.shape
    _, m = w.shape
    return pl.pallas_call(
        _mm_body,
        grid=(n // bm,),
        in_specs=[pl.BlockSpec((bm, k), lambda i: (i, 0)),
                  pl.BlockSpec((k, m), lambda i: (0, 0))],
        out_specs=pl.BlockSpec((bm, m), lambda i: (i, 0)),
        out_shape=jax.ShapeDtypeStruct((n, m), jnp.float32),
    )(x, w)


def _augment_weights(p):
    W = p['W']
    din = W.shape[0]
    cols = [W, (W @ p['att_src'])[:, None], (W @ p['att_dst'])[:, None],
            jnp.zeros((din, AUGW - HID - 2), jnp.float32)]
    return jnp.concatenate(cols, axis=1)


SP_NLOC = 12544
SP_EPAD = 802816
CAP = 25344
NBINS_TOT = 32 * 4 * CAP + EC


@functools.lru_cache(maxsize=None)
def _make_bin_kernel():
    per = SP_EPAD // 32
    nch = per // EC
    mesh = plsc.VectorSubcoreMesh(core_axis_name="c", subcore_axis_name="s")

    @functools.partial(
        pl.kernel, mesh=mesh,
        compiler_params=pltpu.CompilerParams(needs_layout_passes=False,
                                             use_tc_tiling_on_sc=False),
        out_type=(jax.ShapeDtypeStruct((NBINS_TOT,), jnp.int32),
                  jax.ShapeDtypeStruct((NBINS_TOT,), jnp.int32),
                  jax.ShapeDtypeStruct((NBINS_TOT,), jnp.float32),
                  jax.ShapeDtypeStruct((32 * 16,), jnp.int32)),
        scratch_types=[
            pltpu.VMEM((EC,), jnp.int32),
            pltpu.VMEM((EC,), jnp.int32),
            pltpu.VMEM((EC,), jnp.float32),
            pltpu.VMEM((EC,), jnp.int32),
            pltpu.VMEM((EC,), jnp.int32),
            pltpu.VMEM((EC,), jnp.float32),
            pltpu.VMEM((16,), jnp.int32),
            pltpu.SemaphoreType.DMA,
        ],
    )
    def bk(srcp, dstp, eap, bsrc, bdst, bea, bcnt,
           src_v, dst_v, ea_v, pos_v, zi_v, zf_v, cnt_v, sem):
        c = lax.axis_index("c")
        s = lax.axis_index("s")
        t = c * 16 + s
        ar = lax.iota(jnp.int32, LANES)
        zi = jnp.zeros((LANES,), jnp.int32)
        zf = jnp.zeros((LANES,), jnp.float32)
        trash = NBINS_TOT - EC

        def _zb(r, carry):
            o = pl.ds(r * LANES, LANES)
            zi_v[o] = zi
            zf_v[o] = zf
            return carry
        lax.fori_loop(0, EC // LANES, _zb, 0)

        def _chunk(g, fills):
            eb = t * per + g * EC
            pltpu.sync_copy(srcp.at[pl.ds(eb, EC)], src_v)
            pltpu.sync_copy(dstp.at[pl.ds(eb, EC)], dst_v)
            pltpu.sync_copy(eap.at[pl.ds(eb, EC)], ea_v)

            def _grp(j, fills2):
                jo = j * LANES
                d = dst_v[pl.ds(jo, LANES)]
                val = d >= 0
                bid = ((d >= SP_NLOC).astype(jnp.int32)
                       + (d >= 2 * SP_NLOC).astype(jnp.int32)
                       + (d >= 3 * SP_NLOC).astype(jnp.int32))
                pos = trash + ar
                new_fills = []
                for b in range(4):
                    mb = val & (bid == b)
                    mi = mb.astype(jnp.int32)
                    cs = plsc.cumsum(mi)
                    nbb = jnp.sum(mi)
                    posb = (t * 4 + b) * CAP + fills2[b] + cs - 1
                    pos = jnp.where(mb, posb, pos)
                    new_fills.append(fills2[b] + nbb)
                pos_v[pl.ds(jo, LANES)] = pos
                return tuple(new_fills)
            fills = lax.fori_loop(0, EC // LANES, _grp, fills)
            pltpu.sync_copy(src_v, bsrc.at[pos_v])
            pltpu.sync_copy(dst_v, bdst.at[pos_v])
            pltpu.sync_copy(ea_v, bea.at[pos_v])
            return fills
        z = jnp.int32(0)
        fills = lax.fori_loop(0, nch, _chunk, (z, z, z, z))

        cvec = zi
        for b in range(4):
            base = (t * 4 + b) * CAP + fills[b]

            def _zw(k, carry):
                pos_v[pl.ds(k * LANES, LANES)] = base + k * LANES + ar
                return carry
            lax.fori_loop(0, EC // LANES, _zw, 0)
            pltpu.sync_copy(zi_v, bsrc.at[pos_v])
            pltpu.sync_copy(zi_v, bdst.at[pos_v])
            pltpu.sync_copy(zf_v, bea.at[pos_v])
            cvec = jnp.where(ar == b, jnp.broadcast_to(fills[b], (LANES,)),
                             cvec)
        cnt_v[pl.ds(0, LANES)] = cvec
        pltpu.sync_copy(cnt_v, bcnt.at[pl.ds(t * 16, 16)])

    return bk


@functools.lru_cache(maxsize=None)
def _make_binned_layer_kernel(step, use_dst):
    nloc = SP_NLOC
    zc = 196
    nrows = nloc // 16
    nz = nrows // zc
    mesh = plsc.VectorSubcoreMesh(core_axis_name="c", subcore_axis_name="s")

    @functools.partial(
        pl.kernel, mesh=mesh,
        compiler_params=pltpu.CompilerParams(needs_layout_passes=False,
                                             use_tc_tiling_on_sc=False),
        out_type=(jax.ShapeDtypeStruct((2, nloc, HID), jnp.float32),
                  jax.ShapeDtypeStruct((2, nloc, AUXW), jnp.float32)),
        scratch_types=[
            pltpu.VMEM((EC,), jnp.int32),
            pltpu.VMEM((EC,), jnp.int32),
            pltpu.VMEM((EC,), jnp.float32),
            pltpu.VMEM((EC,), jnp.int32),
            pltpu.VMEM((EC,), jnp.float32),
            pltpu.VMEM((EC,), jnp.float32),
            pltpu.VMEM((EC, HID), jnp.float32),
            pltpu.VMEM((EC, AUXW), jnp.float32),
            pltpu.VMEM((32 * 16,), jnp.int32),
            pltpu.VMEM((16,), jnp.float32),
            pltpu.VMEM_SHARED((nloc, HID), jnp.float32),
            pltpu.VMEM_SHARED((nloc, AUXW), jnp.float32),
            pltpu.SemaphoreType.DMA,
        ],
    )
    def ek(h64, bsrc, bdst, bea, bcnt, asrcp, adstp, cvecp, out64, outA,
           src_v, dst_v, ea_v, dstl_v, asrc_c, adst_c,
           rows_v, aux_v, cnt_v, cv_v, acc64, accA, sem):
        c = lax.axis_index("c")
        s = lax.axis_index("s")
        ar = lax.iota(jnp.int32, LANES)
        zf = jnp.zeros((LANES,), jnp.float32)
        is0, is1, is2 = (ar == 0), (ar == 1), (ar == 2)
        q = 2 * step + c
        dst_lo = q * nloc

        def _zr(r, carry):
            for k in range(HID // LANES):
                rows_v[r, pl.ds(k * LANES, LANES)] = zf
            aux_v[r, pl.ds(0, LANES)] = zf
            return carry
        lax.fori_loop(0, zc, _zr, 0)
        r0 = s * nrows
        for z in range(nz):
            pltpu.sync_copy(rows_v.at[pl.ds(0, zc)],
                            acc64.at[pl.ds(r0 + z * zc, zc)])
            pltpu.sync_copy(aux_v.at[pl.ds(0, zc)],
                            accA.at[pl.ds(r0 + z * zc, zc)])
        pltpu.sync_copy(bcnt, cnt_v)
        pltpu.sync_copy(cvecp, cv_v)
        plsc.subcore_barrier()
        cvec = cv_v[pl.ds(0, LANES)]

        for bi in range(2):
            t = 2 * s + bi
            cnt16 = cnt_v[pl.ds(t * 16, LANES)]
            nb = cnt16.at[ar - ar + q].get(mode='promise_in_bounds')[0]
            nch = (nb + EC - 1) // EC
            bin_base = (t * 4 + q) * CAP

            def _chunk(g, carry):
                eb = bin_base + g * EC
                pltpu.sync_copy(bsrc.at[pl.ds(eb, EC)], src_v)
                pltpu.sync_copy(bdst.at[pl.ds(eb, EC)], dst_v)
                pltpu.sync_copy(bea.at[pl.ds(eb, EC)], ea_v)
                pltpu.async_copy(h64.at[src_v], rows_v, sem).wait()
                pltpu.async_copy(asrcp.at[src_v], asrc_c, sem).wait()
                if use_dst:
                    pltpu.async_copy(adstp.at[dst_v], adst_c, sem).wait()

                def _grpB(j, carry2):
                    jo = j * LANES
                    dst16 = dst_v[pl.ds(jo, LANES)]
                    ea16 = ea_v[pl.ds(jo, LANES)]
                    m = (g * EC + jo + ar) < nb
                    dstl = dst16 - dst_lo
                    dstl_v[pl.ds(jo, LANES)] = jnp.where(m, dstl, 0)
                    a = asrc_c[pl.ds(jo, LANES)] + ea16 * cvec
                    if use_dst:
                        a = a + adst_c[pl.ds(jo, LANES)]
                    a = jnp.where(a >= 0.0, a, 0.2 * a)
                    ex = jnp.where(m, jnp.exp(a), 0.0)
                    one = jnp.where(m, 1.0, 0.0)
                    eam = jnp.where(m, ea16 * cvec, 0.0)
                    for jj in range(LANES):
                        sel = ar - ar + jj
                        bex = ex.at[sel].get(mode='promise_in_bounds')
                        bea16 = eam.at[sel].get(mode='promise_in_bounds')
                        bone = one.at[sel].get(mode='promise_in_bounds')
                        aux = jnp.where(is0, bex,
                                        jnp.where(is1, bea16,
                                                  jnp.where(is2, bone, zf)))
                        aux_v[jo + jj, pl.ds(0, LANES)] = aux
                        for k in range(HID // LANES):
                            sl = pl.ds(k * LANES, LANES)
                            rows_v[jo + jj, sl] = rows_v[jo + jj, sl] * bex
                    return carry2
                lax.fori_loop(0, EC // LANES, _grpB, 0)
                pltpu.sync_copy(rows_v, acc64.at[dstl_v], add=True)
                pltpu.sync_copy(aux_v, accA.at[dstl_v], add=True)
                return carry
            lax.fori_loop(0, nch, _chunk, 0)
        plsc.subcore_barrier()
        pltpu.sync_copy(acc64.at[pl.ds(r0, nrows)],
                        out64.at[c, pl.ds(r0, nrows)])
        pltpu.sync_copy(accA.at[pl.ds(r0, nrows)],
                        outA.at[c, pl.ds(r0, nrows)])

    return ek


@functools.lru_cache(maxsize=None)
def _make_edge_kernel(e_pad, nloc, zc, split, use_dst, step):
    nrows = nloc // 16
    nz = nrows // zc
    assert nrows % zc == 0 and zc <= EC and nloc % 16 == 0
    per = e_pad // 16 if split else e_pad // 32
    nch = per // EC
    assert per % EC == 0

    mesh = plsc.VectorSubcoreMesh(core_axis_name="c", subcore_axis_name="s")

    @functools.partial(
        pl.kernel, mesh=mesh,
        compiler_params=pltpu.CompilerParams(needs_layout_passes=False,
                                             use_tc_tiling_on_sc=False),
        out_type=(jax.ShapeDtypeStruct((2, nloc, HID), jnp.float32),
                  jax.ShapeDtypeStruct((2, nloc, AUXW), jnp.float32)),
        scratch_types=[
            pltpu.VMEM((EC,), jnp.int32),
            pltpu.VMEM((EC,), jnp.int32),
            pltpu.VMEM((EC,), jnp.float32),
            pltpu.VMEM((EC,), jnp.int32),
            pltpu.VMEM((EC,), jnp.int32),
            pltpu.VMEM((EC,), jnp.float32),
            pltpu.VMEM((EC,), jnp.float32),
            pltpu.VMEM((EC, HID), jnp.float32),
            pltpu.VMEM((EC, AUXW), jnp.float32),
            pltpu.VMEM_SHARED((nloc, HID), jnp.float32),
            pltpu.VMEM_SHARED((nloc, AUXW), jnp.float32),
            pltpu.SemaphoreType.DMA,
        ],
    )
    def ek(h64, srcp, dstp, eap, asrcp, adstp, out64, outA,
           src_v, dst_v, ea_v, dstl_v, dstg_v, asrc_c, adst_c,
           rows_v, aux_v, acc64, accA, sem):
        c = lax.axis_index("c")
        s = lax.axis_index("s")
        ar = lax.iota(jnp.int32, LANES)
        zf = jnp.zeros((LANES,), jnp.float32)
        is0, is1, is2 = (ar == 0), (ar == 1), (ar == 2)

        def _zr(r, carry):
            for k in range(HID // LANES):
                rows_v[r, pl.ds(k * LANES, LANES)] = zf
            aux_v[r, pl.ds(0, LANES)] = zf
            return carry
        lax.fori_loop(0, zc, _zr, 0)
        r0 = s * nrows
        for z in range(nz):
            pltpu.sync_copy(rows_v.at[pl.ds(0, zc)],
                            acc64.at[pl.ds(r0 + z * zc, zc)])
            pltpu.sync_copy(aux_v.at[pl.ds(0, zc)],
                            accA.at[pl.ds(r0 + z * zc, zc)])
        plsc.subcore_barrier()

        dst_lo = (2 * step + c) * nloc if split else 0
        base = s * per if split else (s * 2 + c) * per

        def _chunk(g, carry):
            eb = base + g * EC
            pltpu.sync_copy(srcp.at[pl.ds(eb, EC)], src_v)
            pltpu.sync_copy(dstp.at[pl.ds(eb, EC)], dst_v)
            pltpu.sync_copy(eap.at[pl.ds(eb, EC)], ea_v)
            pltpu.async_copy(h64.at[src_v], rows_v, sem).wait()
            pltpu.async_copy(asrcp.at[src_v], asrc_c, sem).wait()

            def _grpA(j, carry2):
                jo = j * LANES
                dst16 = dst_v[pl.ds(jo, LANES)]
                dstl = dst16 - dst_lo
                m = (dstl >= 0) & (dstl < nloc)
                dstl_v[pl.ds(jo, LANES)] = jnp.where(m, dstl, 0)
                dstg_v[pl.ds(jo, LANES)] = jnp.where(m, dst16, 0)
                return carry2
            lax.fori_loop(0, EC // LANES, _grpA, 0)
            if use_dst:
                pltpu.async_copy(adstp.at[dstg_v], adst_c, sem).wait()

            def _grpB(j, carry2):
                jo = j * LANES
                dstl = dstl_v[pl.ds(jo, LANES)]
                dst16 = dst_v[pl.ds(jo, LANES)]
                ea16 = ea_v[pl.ds(jo, LANES)]
                m = (dst16 - dst_lo >= 0) & (dst16 - dst_lo < nloc)
                a = asrc_c[pl.ds(jo, LANES)] + ea16
                if use_dst:
                    a = a + adst_c[pl.ds(jo, LANES)]
                a = jnp.where(a >= 0.0, a, 0.2 * a)
                ex = jnp.where(m, jnp.exp(a), 0.0)
                one = jnp.where(m, 1.0, 0.0)
                eam = jnp.where(m, ea16, 0.0)
                for jj in range(LANES):
                    sel = ar - ar + jj
                    bex = ex.at[sel].get(mode='promise_in_bounds')
                    bea = eam.at[sel].get(mode='promise_in_bounds')
                    bone = one.at[sel].get(mode='promise_in_bounds')
                    aux = jnp.where(is0, bex,
                                    jnp.where(is1, bea,
                                              jnp.where(is2, bone, zf)))
                    aux_v[jo + jj, pl.ds(0, LANES)] = aux
                    for k in range(HID // LANES):
                        sl = pl.ds(k * LANES, LANES)
                        rows_v[jo + jj, sl] = rows_v[jo + jj, sl] * bex
                return carry2
            lax.fori_loop(0, EC // LANES, _grpB, 0)
            pltpu.sync_copy(rows_v, acc64.at[dstl_v], add=True)
            pltpu.sync_copy(aux_v, accA.at[dstl_v], add=True)
            return carry
        lax.fori_loop(0, nch, _chunk, 0)
        plsc.subcore_barrier()
        pltpu.sync_copy(acc64.at[pl.ds(r0, nrows)],
                        out64.at[c, pl.ds(r0, nrows)])
        pltpu.sync_copy(accA.at[pl.ds(r0, nrows)],
                        outA.at[c, pl.ds(r0, nrows)])

    return ek


def _pad1(a, n, fill):
    return jnp.pad(a, (0, n - a.shape[0]), constant_values=fill)


def _edge_pass_sc(h64, src_p, dst_p, ea_p, asrc_arr, adst_arr, num_dst,
                  nloc, zc, split, use_dst):
    n_steps = 2 if split else 1
    p64, pA = [], []
    for step in range(n_steps):
        ek = _make_edge_kernel(src_p.shape[0], nloc, zc, split, use_dst,
                               step)
        o64, oA = ek(h64, src_p, dst_p, ea_p, asrc_arr, adst_arr)
        p64 += [o64[0], o64[1]]
        pA += [oA[0], oA[1]]
    if split:
        return (jnp.concatenate(p64, axis=0)[:num_dst],
                jnp.concatenate(pA, axis=0)[:num_dst])
    return (p64[0] + p64[1])[:num_dst], (pA[0] + pA[1])[:num_dst]


def _edge_pass_binned(h64, bins, asrc_arr, adst_arr, cval, num_dst, use_dst):
    bsrc, bdst, bea, bcnt = bins
    cvec = jnp.full((16,), cval, jnp.float32)
    p64, pA = [], []
    for step in range(2):
        ek = _make_binned_layer_kernel(step, use_dst)
        o64, oA = ek(h64, bsrc, bdst, bea, bcnt, asrc_arr, adst_arr, cvec)
        p64 += [o64[0], o64[1]]
        pA += [oA[0], oA[1]]
    return (jnp.concatenate(p64, axis=0)[:num_dst],
            jnp.concatenate(pA, axis=0)[:num_dst])


def _gat_layer(x, p, src_p, dst_p, ea_raw_p, num_dst, self_loops, use_dst,
               nloc, zc, split, bins=None):
    c = jnp.dot(p['W_e'][0], p['att_edge'])
    ea_p = c * ea_raw_p
    Waug = _augment_weights(p)
    H = _matmul(x, Waug)
    h64 = H[:, :HID]
    asrc_arr = H[:, HID]
    adst_arr = H[:, HID + 1] if use_dst else jnp.zeros((16,), jnp.float32)
    if bins is not None:
        num, stats = _edge_pass_binned(h64, bins, asrc_arr, adst_arr, c,
                                       num_dst, use_dst)
    else:
        num, stats = _edge_pass_sc(h64, src_p, dst_p, ea_p, asrc_arr,
                                   adst_arr, num_dst, nloc, zc, split,
                                   use_dst)
    exsum, easum, cnt = stats[:, 0], stats[:, 1], stats[:, 2]
    if self_loops:
        loop_ea = easum / jnp.maximum(cnt, 1.0)
        a_loop = asrc_arr + adst_arr + loop_ea
        a_loop = jnp.where(a_loop >= 0, a_loop, 0.2 * a_loop)
        ex_loop = jnp.exp(a_loop)
        num = num + ex_loop[:, None] * h64
        denom = exsum + ex_loop + 1e-16
    else:
        denom = exsum + 1e-16
    return num / denom[:, None] + p['b']


def kernel(x_spatial, global_data, edge_index_spatial, edge_attr_spatial,
           bip_edge_index, bip_edge_attr, x_species, x_species_phylo,
           edge_index_species, edge_attr_species, params):
    n_space = x_spatial.shape[0]
    n_species = x_species.shape[0]

    def _prep(src, dst, ea, e_pad):
        return (_pad1(src, e_pad, 0), _pad1(dst, e_pad, -1),
                _pad1(ea[:, 0], e_pad, 0.0))

    sp_e = _prep(edge_index_spatial[0], edge_index_spatial[1],
                 edge_attr_spatial, SP_EPAD)
    bp_e = _prep(bip_edge_index[0], bip_edge_index[1],
                 bip_edge_attr, 507904)
    pc_e = _prep(edge_index_species[0], edge_index_species[1],
                 edge_attr_species, 163840)

    SPG = dict(nloc=SP_NLOC, zc=196, split=True)
    SCG = dict(nloc=10000, zc=125, split=False)

    bins = _make_bin_kernel()(*sp_e)

    si = jnp.concatenate([x_spatial, global_data], axis=1)
    h = jax.nn.relu(_gat_layer(si, params['space0'], *sp_e, n_space,
                               True, True, bins=bins, **SPG))
    h = _gat_layer(h, params['space1'], *sp_e, n_space, True, True,
                   bins=bins, **SPG)
    h = jax.nn.relu(h)
    s2s = _gat_layer(h, params['bip'], *bp_e, n_species, False, False, **SCG)
    spin = jnp.concatenate([s2s, x_species, x_species_phylo], axis=1)
    g = jax.nn.relu(_gat_layer(spin, params['sp0'], *pc_e, n_species,
                               True, True, **SCG))
    g = _gat_layer(g, params['sp1'], *pc_e, n_species, True, True, **SCG)
    g = jax.nn.relu(g)
    return _matmul(g, params['fc_W']) + params['fc_b']


# speedup vs baseline: 20.0825x; 2.0658x over previous
import functools

import jax
import jax.numpy as jnp
import numpy as np
from jax import lax
from jax.experimental import pallas as pl
from jax.experimental.pallas import tpu as pltpu
from jax.experimental.pallas import tpu_sc as plsc

HID = 64
AUGW = 80
AUXW = 16
LANES = 16
EC = 256


def _mm_body(x_ref, w_ref, o_ref):
    o_ref[...] = jnp.dot(x_ref[...], w_ref[...],
                         preferred_element_type=jnp.float32)


def _matmul(x, w, bm=400):
    n, k = x.shape
    _, m = w.shape
    return pl.pallas_call(
        _mm_body,
        grid=(n // bm,),
        in_specs=[pl.BlockSpec((bm, k), lambda i: (i, 0)),
                  pl.BlockSpec((k, m), lambda i: (0, 0))],
        out_specs=pl.BlockSpec((bm, m), lambda i: (i, 0)),
        out_shape=jax.ShapeDtypeStruct((n, m), jnp.float32),
    )(x, w)


def _augment_weights(p):
    W = p['W']
    din = W.shape[0]
    cols = [W, (W @ p['att_src'])[:, None], (W @ p['att_dst'])[:, None],
            jnp.zeros((din, AUGW - HID - 2), jnp.float32)]
    return jnp.concatenate(cols, axis=1)


SP_NLOC = 12544
SP_EPAD = 802816
CAP = 25344
NBINS_TOT = 32 * 4 * CAP
SB = 2 * EC


@functools.lru_cache(maxsize=None)
def _make_bin_kernel():
    per = SP_EPAD // 32
    nch = per // EC
    mesh = plsc.VectorSubcoreMesh(core_axis_name="c", subcore_axis_name="s")

    @functools.partial(
        pl.kernel, mesh=mesh,
        compiler_params=pltpu.CompilerParams(needs_layout_passes=False,
                                             use_tc_tiling_on_sc=False),
        out_type=(jax.ShapeDtypeStruct((NBINS_TOT,), jnp.int32),
                  jax.ShapeDtypeStruct((NBINS_TOT,), jnp.int32),
                  jax.ShapeDtypeStruct((NBINS_TOT,), jnp.float32),
                  jax.ShapeDtypeStruct((32 * 16,), jnp.int32)),
        scratch_types=(
            [pltpu.VMEM((EC,), jnp.int32),
             pltpu.VMEM((EC,), jnp.int32),
             pltpu.VMEM((EC,), jnp.float32)]
            + [pltpu.VMEM((SB,), jnp.int32) for _ in range(4)]
            + [pltpu.VMEM((SB,), jnp.int32) for _ in range(4)]
            + [pltpu.VMEM((SB,), jnp.float32) for _ in range(4)]
            + [pltpu.VMEM((16,), jnp.int32),
               pltpu.SemaphoreType.DMA]
        ),
    )
    def bk(srcp, dstp, eap, bsrc, bdst, bea, bcnt,
           src_v, dst_v, ea_v,
           ss0, ss1, ss2, ss3, ds0, ds1, ds2, ds3, es0, es1, es2, es3,
           cnt_v, sem):
        c = lax.axis_index("c")
        s = lax.axis_index("s")
        t = c * 16 + s
        ar = lax.iota(jnp.int32, LANES)
        zi = jnp.zeros((LANES,), jnp.int32)
        zf = jnp.zeros((LANES,), jnp.float32)
        stages = [(ss0, ds0, es0), (ss1, ds1, es1),
                  (ss2, ds2, es2), (ss3, ds3, es3)]

        def _chunk(g, carry):
            fills, offs = carry[:4], carry[4:]
            eb = pl.multiple_of(t * per + g * EC, 8)
            pltpu.sync_copy(srcp.at[pl.ds(eb, EC)], src_v)
            pltpu.sync_copy(dstp.at[pl.ds(eb, EC)], dst_v)
            pltpu.sync_copy(eap.at[pl.ds(eb, EC)], ea_v)

            def _grp(j, fills2):
                jo = j * LANES
                sv = src_v[pl.ds(jo, LANES)]
                d = dst_v[pl.ds(jo, LANES)]
                ev = ea_v[pl.ds(jo, LANES)]
                val = d >= 0
                bid = ((d >= SP_NLOC).astype(jnp.int32)
                       + (d >= 2 * SP_NLOC).astype(jnp.int32)
                       + (d >= 3 * SP_NLOC).astype(jnp.int32))
                new_fills = []
                for b in range(4):
                    mb = val & (bid == b)
                    mi = mb.astype(jnp.int32)
                    pos = fills2[b] + plsc.cumsum(mi) - 1
                    sb, db, eab = stages[b]
                    plsc.store_scatter(sb, [pos], sv, mask=mb)
                    plsc.store_scatter(db, [pos], d, mask=mb)
                    plsc.store_scatter(eab, [pos], ev, mask=mb)
                    new_fills.append(fills2[b] + jnp.sum(mi))
                return tuple(new_fills)
            fills = lax.fori_loop(0, EC // LANES, _grp, tuple(fills))

            new_carry = []
            for b in range(4):
                sb, db, eab = stages[b]
                do_flush = fills[b] >= EC
                hoff = pl.multiple_of((t * 4 + b) * CAP + offs[b], 8)

                @pl.when(do_flush)
                def _flush(sb=sb, db=db, eab=eab, hoff=hoff):
                    pltpu.sync_copy(sb.at[pl.ds(0, EC)],
                                    bsrc.at[pl.ds(hoff, EC)])
                    pltpu.sync_copy(db.at[pl.ds(0, EC)],
                                    bdst.at[pl.ds(hoff, EC)])
                    pltpu.sync_copy(eab.at[pl.ds(0, EC)],
                                    bea.at[pl.ds(hoff, EC)])
                    for k in range(EC // LANES):
                        o = pl.ds(k * LANES, LANES)
                        o2 = pl.ds(EC + k * LANES, LANES)
                        sb[o] = sb[o2]
                        db[o] = db[o2]
                        eab[o] = eab[o2]
                new_carry.append(jnp.where(do_flush, fills[b] - EC,
                                           fills[b]))
            for b in range(4):
                new_carry.append(offs[b]
                                 + jnp.where(fills[b] >= EC, EC, 0))
            return tuple(new_carry)
        z = jnp.int32(0)
        carry = lax.fori_loop(0, nch, _chunk, (z,) * 8)
        fills, offs = carry[:4], carry[4:]

        cvec = zi
        for b in range(4):
            sb, db, eab = stages[b]
            fill = fills[b]
            for k in range(EC // LANES):
                pos = k * LANES + ar
                mz = pos >= fill
                plsc.store_scatter(sb, [pos], zi, mask=mz)
                plsc.store_scatter(db, [pos], zi, mask=mz)
                plsc.store_scatter(eab, [pos], zf, mask=mz)
            hoff = pl.multiple_of((t * 4 + b) * CAP + offs[b], 8)
            pltpu.sync_copy(sb.at[pl.ds(0, EC)], bsrc.at[pl.ds(hoff, EC)])
            pltpu.sync_copy(db.at[pl.ds(0, EC)], bdst.at[pl.ds(hoff, EC)])
            pltpu.sync_copy(eab.at[pl.ds(0, EC)], bea.at[pl.ds(hoff, EC)])
            cvec = jnp.where(ar == b,
                             jnp.broadcast_to(offs[b] + fill, (LANES,)),
                             cvec)
        cnt_v[pl.ds(0, LANES)] = cvec
        pltpu.sync_copy(cnt_v, bcnt.at[pl.ds(pl.multiple_of(t * 16, 8), 16)])

    return bk


@functools.lru_cache(maxsize=None)
def _make_binned_layer_kernel(step, use_dst):
    nloc = SP_NLOC
    zc = 196
    nrows = nloc // 16
    nz = nrows // zc
    mesh = plsc.VectorSubcoreMesh(core_axis_name="c", subcore_axis_name="s")

    @functools.partial(
        pl.kernel, mesh=mesh,
        compiler_params=pltpu.CompilerParams(needs_layout_passes=False,
                                             use_tc_tiling_on_sc=False),
        out_type=(jax.ShapeDtypeStruct((2, nloc, HID), jnp.float32),
                  jax.ShapeDtypeStruct((2, nloc, AUXW), jnp.float32)),
        scratch_types=[
            pltpu.VMEM((EC,), jnp.int32),
            pltpu.VMEM((EC,), jnp.int32),
            pltpu.VMEM((EC,), jnp.float32),
            pltpu.VMEM((EC,), jnp.int32),
            pltpu.VMEM((EC,), jnp.float32),
            pltpu.VMEM((EC,), jnp.float32),
            pltpu.VMEM((EC, HID), jnp.float32),
            pltpu.VMEM((EC, AUXW), jnp.float32),
            pltpu.VMEM((32 * 16,), jnp.int32),
            pltpu.VMEM((16,), jnp.float32),
            pltpu.VMEM_SHARED((nloc, HID), jnp.float32),
            pltpu.VMEM_SHARED((nloc, AUXW), jnp.float32),
            pltpu.SemaphoreType.DMA,
        ],
    )
    def ek(h64, bsrc, bdst, bea, bcnt, asrcp, adstp, cvecp, out64, outA,
           src_v, dst_v, ea_v, dstl_v, asrc_c, adst_c,
           rows_v, aux_v, cnt_v, cv_v, acc64, accA, sem):
        c = lax.axis_index("c")
        s = lax.axis_index("s")
        ar = lax.iota(jnp.int32, LANES)
        zf = jnp.zeros((LANES,), jnp.float32)
        is0, is1, is2 = (ar == 0), (ar == 1), (ar == 2)
        q = 2 * step + c
        dst_lo = q * nloc

        def _zr(r, carry):
            for k in range(HID // LANES):
                rows_v[r, pl.ds(k * LANES, LANES)] = zf
            aux_v[r, pl.ds(0, LANES)] = zf
            return carry
        lax.fori_loop(0, zc, _zr, 0)
        r0 = s * nrows
        for z in range(nz):
            pltpu.sync_copy(rows_v.at[pl.ds(0, zc)],
                            acc64.at[pl.ds(r0 + z * zc, zc)])
            pltpu.sync_copy(aux_v.at[pl.ds(0, zc)],
                            accA.at[pl.ds(r0 + z * zc, zc)])
        pltpu.sync_copy(bcnt, cnt_v)
        pltpu.sync_copy(cvecp, cv_v)
        plsc.subcore_barrier()
        cvec = cv_v[pl.ds(0, LANES)]

        for bi in range(2):
            t = 2 * s + bi
            cnt16 = cnt_v[pl.ds(t * 16, LANES)]
            nb = cnt16.at[ar - ar + q].get(mode='promise_in_bounds')[0]
            nch = (nb + EC - 1) // EC
            bin_base = (t * 4 + q) * CAP

            def _chunk(g, carry):
                eb = pl.multiple_of(bin_base + g * EC, 8)
                pltpu.sync_copy(bsrc.at[pl.ds(eb, EC)], src_v)
                pltpu.sync_copy(bdst.at[pl.ds(eb, EC)], dst_v)
                pltpu.sync_copy(bea.at[pl.ds(eb, EC)], ea_v)
                pltpu.async_copy(h64.at[src_v], rows_v, sem).wait()
                pltpu.async_copy(asrcp.at[src_v], asrc_c, sem).wait()
                if use_dst:
                    pltpu.async_copy(adstp.at[dst_v], adst_c, sem).wait()

                def _grpB(j, carry2):
                    jo = j * LANES
                    dst16 = dst_v[pl.ds(jo, LANES)]
                    ea16 = ea_v[pl.ds(jo, LANES)]
                    m = (g * EC + jo + ar) < nb
                    dstl = dst16 - dst_lo
                    dstl_v[pl.ds(jo, LANES)] = jnp.where(m, dstl, 0)
                    a = asrc_c[pl.ds(jo, LANES)] + ea16 * cvec
                    if use_dst:
                        a = a + adst_c[pl.ds(jo, LANES)]
                    a = jnp.where(a >= 0.0, a, 0.2 * a)
                    ex = jnp.where(m, jnp.exp(a), 0.0)
                    one = jnp.where(m, 1.0, 0.0)
                    eam = jnp.where(m, ea16 * cvec, 0.0)
                    for jj in range(LANES):
                        sel = ar - ar + jj
                        bex = ex.at[sel].get(mode='promise_in_bounds')
                        bea16 = eam.at[sel].get(mode='promise_in_bounds')
                        bone = one.at[sel].get(mode='promise_in_bounds')
                        aux = jnp.where(is0, bex,
                                        jnp.where(is1, bea16,
                                                  jnp.where(is2, bone, zf)))
                        aux_v[jo + jj, pl.ds(0, LANES)] = aux
                        for k in range(HID // LANES):
                            sl = pl.ds(k * LANES, LANES)
                            rows_v[jo + jj, sl] = rows_v[jo + jj, sl] * bex
                    return carry2
                lax.fori_loop(0, EC // LANES, _grpB, 0)
                pltpu.sync_copy(rows_v, acc64.at[dstl_v], add=True)
                pltpu.sync_copy(aux_v, accA.at[dstl_v], add=True)
                return carry
            lax.fori_loop(0, nch, _chunk, 0)
        plsc.subcore_barrier()
        pltpu.sync_copy(acc64.at[pl.ds(r0, nrows)],
                        out64.at[c, pl.ds(r0, nrows)])
        pltpu.sync_copy(accA.at[pl.ds(r0, nrows)],
                        outA.at[c, pl.ds(r0, nrows)])

    return ek


@functools.lru_cache(maxsize=None)
def _make_edge_kernel(e_pad, nloc, zc, split, use_dst, step):
    nrows = nloc // 16
    nz = nrows // zc
    assert nrows % zc == 0 and zc <= EC and nloc % 16 == 0
    per = e_pad // 16 if split else e_pad // 32
    nch = per // EC
    assert per % EC == 0

    mesh = plsc.VectorSubcoreMesh(core_axis_name="c", subcore_axis_name="s")

    @functools.partial(
        pl.kernel, mesh=mesh,
        compiler_params=pltpu.CompilerParams(needs_layout_passes=False,
                                             use_tc_tiling_on_sc=False),
        out_type=(jax.ShapeDtypeStruct((2, nloc, HID), jnp.float32),
                  jax.ShapeDtypeStruct((2, nloc, AUXW), jnp.float32)),
        scratch_types=[
            pltpu.VMEM((EC,), jnp.int32),
            pltpu.VMEM((EC,), jnp.int32),
            pltpu.VMEM((EC,), jnp.float32),
            pltpu.VMEM((EC,), jnp.int32),
            pltpu.VMEM((EC,), jnp.int32),
            pltpu.VMEM((EC,), jnp.float32),
            pltpu.VMEM((EC,), jnp.float32),
            pltpu.VMEM((EC, HID), jnp.float32),
            pltpu.VMEM((EC, AUXW), jnp.float32),
            pltpu.VMEM_SHARED((nloc, HID), jnp.float32),
            pltpu.VMEM_SHARED((nloc, AUXW), jnp.float32),
            pltpu.SemaphoreType.DMA,
        ],
    )
    def ek(h64, srcp, dstp, eap, asrcp, adstp, out64, outA,
           src_v, dst_v, ea_v, dstl_v, dstg_v, asrc_c, adst_c,
           rows_v, aux_v, acc64, accA, sem):
        c = lax.axis_index("c")
        s = lax.axis_index("s")
        ar = lax.iota(jnp.int32, LANES)
        zf = jnp.zeros((LANES,), jnp.float32)
        is0, is1, is2 = (ar == 0), (ar == 1), (ar == 2)

        def _zr(r, carry):
            for k in range(HID // LANES):
                rows_v[r, pl.ds(k * LANES, LANES)] = zf
            aux_v[r, pl.ds(0, LANES)] = zf
            return carry
        lax.fori_loop(0, zc, _zr, 0)
        r0 = s * nrows
        for z in range(nz):
            pltpu.sync_copy(rows_v.at[pl.ds(0, zc)],
                            acc64.at[pl.ds(r0 + z * zc, zc)])
            pltpu.sync_copy(aux_v.at[pl.ds(0, zc)],
                            accA.at[pl.ds(r0 + z * zc, zc)])
        plsc.subcore_barrier()

        dst_lo = (2 * step + c) * nloc if split else 0
        base = s * per if split else (s * 2 + c) * per

        def _chunk(g, carry):
            eb = base + g * EC
            pltpu.sync_copy(srcp.at[pl.ds(eb, EC)], src_v)
            pltpu.sync_copy(dstp.at[pl.ds(eb, EC)], dst_v)
            pltpu.sync_copy(eap.at[pl.ds(eb, EC)], ea_v)
            pltpu.async_copy(h64.at[src_v], rows_v, sem).wait()
            pltpu.async_copy(asrcp.at[src_v], asrc_c, sem).wait()

            def _grpA(j, carry2):
                jo = j * LANES
                dst16 = dst_v[pl.ds(jo, LANES)]
                dstl = dst16 - dst_lo
                m = (dstl >= 0) & (dstl < nloc)
                dstl_v[pl.ds(jo, LANES)] = jnp.where(m, dstl, 0)
                dstg_v[pl.ds(jo, LANES)] = jnp.where(m, dst16, 0)
                return carry2
            lax.fori_loop(0, EC // LANES, _grpA, 0)
            if use_dst:
                pltpu.async_copy(adstp.at[dstg_v], adst_c, sem).wait()

            def _grpB(j, carry2):
                jo = j * LANES
                dstl = dstl_v[pl.ds(jo, LANES)]
                dst16 = dst_v[pl.ds(jo, LANES)]
                ea16 = ea_v[pl.ds(jo, LANES)]
                m = (dst16 - dst_lo >= 0) & (dst16 - dst_lo < nloc)
                a = asrc_c[pl.ds(jo, LANES)] + ea16
                if use_dst:
                    a = a + adst_c[pl.ds(jo, LANES)]
                a = jnp.where(a >= 0.0, a, 0.2 * a)
                ex = jnp.where(m, jnp.exp(a), 0.0)
                one = jnp.where(m, 1.0, 0.0)
                eam = jnp.where(m, ea16, 0.0)
                for jj in range(LANES):
                    sel = ar - ar + jj
                    bex = ex.at[sel].get(mode='promise_in_bounds')
                    bea = eam.at[sel].get(mode='promise_in_bounds')
                    bone = one.at[sel].get(mode='promise_in_bounds')
                    aux = jnp.where(is0, bex,
                                    jnp.where(is1, bea,
                                              jnp.where(is2, bone, zf)))
                    aux_v[jo + jj, pl.ds(0, LANES)] = aux
                    for k in range(HID // LANES):
                        sl = pl.ds(k * LANES, LANES)
                        rows_v[jo + jj, sl] = rows_v[jo + jj, sl] * bex
                return carry2
            lax.fori_loop(0, EC // LANES, _grpB, 0)
            pltpu.sync_copy(rows_v, acc64.at[dstl_v], add=True)
            pltpu.sync_copy(aux_v, accA.at[dstl_v], add=True)
            return carry
        lax.fori_loop(0, nch, _chunk, 0)
        plsc.subcore_barrier()
        pltpu.sync_copy(acc64.at[pl.ds(r0, nrows)],
                        out64.at[c, pl.ds(r0, nrows)])
        pltpu.sync_copy(accA.at[pl.ds(r0, nrows)],
                        outA.at[c, pl.ds(r0, nrows)])

    return ek


def _pad1(a, n, fill):
    return jnp.pad(a, (0, n - a.shape[0]), constant_values=fill)


def _edge_pass_sc(h64, src_p, dst_p, ea_p, asrc_arr, adst_arr, num_dst,
                  nloc, zc, split, use_dst):
    n_steps = 2 if split else 1
    p64, pA = [], []
    for step in range(n_steps):
        ek = _make_edge_kernel(src_p.shape[0], nloc, zc, split, use_dst,
                               step)
        o64, oA = ek(h64, src_p, dst_p, ea_p, asrc_arr, adst_arr)
        p64 += [o64[0], o64[1]]
        pA += [oA[0], oA[1]]
    if split:
        return (jnp.concatenate(p64, axis=0)[:num_dst],
                jnp.concatenate(pA, axis=0)[:num_dst])
    return (p64[0] + p64[1])[:num_dst], (pA[0] + pA[1])[:num_dst]


def _edge_pass_binned(h64, bins, asrc_arr, adst_arr, cval, num_dst, use_dst):
    bsrc, bdst, bea, bcnt = bins
    cvec = jnp.full((16,), cval, jnp.float32)
    p64, pA = [], []
    for step in range(2):
        ek = _make_binned_layer_kernel(step, use_dst)
        o64, oA = ek(h64, bsrc, bdst, bea, bcnt, asrc_arr, adst_arr, cvec)
        p64 += [o64[0], o64[1]]
        pA += [oA[0], oA[1]]
    return (jnp.concatenate(p64, axis=0)[:num_dst],
            jnp.concatenate(pA, axis=0)[:num_dst])


def _gat_layer(x, p, src_p, dst_p, ea_raw_p, num_dst, self_loops, use_dst,
               nloc, zc, split, bins=None):
    c = jnp.dot(p['W_e'][0], p['att_edge'])
    ea_p = c * ea_raw_p
    Waug = _augment_weights(p)
    H = _matmul(x, Waug)
    h64 = H[:, :HID]
    asrc_arr = H[:, HID]
    adst_arr = H[:, HID + 1] if use_dst else jnp.zeros((16,), jnp.float32)
    if bins is not None:
        num, stats = _edge_pass_binned(h64, bins, asrc_arr, adst_arr, c,
                                       num_dst, use_dst)
    else:
        num, stats = _edge_pass_sc(h64, src_p, dst_p, ea_p, asrc_arr,
                                   adst_arr, num_dst, nloc, zc, split,
                                   use_dst)
    exsum, easum, cnt = stats[:, 0], stats[:, 1], stats[:, 2]
    if self_loops:
        loop_ea = easum / jnp.maximum(cnt, 1.0)
        a_loop = asrc_arr + adst_arr + loop_ea
        a_loop = jnp.where(a_loop >= 0, a_loop, 0.2 * a_loop)
        ex_loop = jnp.exp(a_loop)
        num = num + ex_loop[:, None] * h64
        denom = exsum + ex_loop + 1e-16
    else:
        denom = exsum + 1e-16
    return num / denom[:, None] + p['b']


def kernel(x_spatial, global_data, edge_index_spatial, edge_attr_spatial,
           bip_edge_index, bip_edge_attr, x_species, x_species_phylo,
           edge_index_species, edge_attr_species, params):
    n_space = x_spatial.shape[0]
    n_species = x_species.shape[0]

    def _prep(src, dst, ea, e_pad):
        return (_pad1(src, e_pad, 0), _pad1(dst, e_pad, -1),
                _pad1(ea[:, 0], e_pad, 0.0))

    sp_e = _prep(edge_index_spatial[0], edge_index_spatial[1],
                 edge_attr_spatial, SP_EPAD)
    bp_e = _prep(bip_edge_index[0], bip_edge_index[1],
                 bip_edge_attr, 507904)
    pc_e = _prep(edge_index_species[0], edge_index_species[1],
                 edge_attr_species, 163840)

    SPG = dict(nloc=SP_NLOC, zc=196, split=True)
    SCG = dict(nloc=10000, zc=125, split=False)

    bins = _make_bin_kernel()(*sp_e)

    si = jnp.concatenate([x_spatial, global_data], axis=1)
    h = jax.nn.relu(_gat_layer(si, params['space0'], *sp_e, n_space,
                               True, True, bins=bins, **SPG))
    h = _gat_layer(h, params['space1'], *sp_e, n_space, True, True,
                   bins=bins, **SPG)
    h = jax.nn.relu(h)
    s2s = _gat_layer(h, params['bip'], *bp_e, n_species, False, False, **SCG)
    spin = jnp.concatenate([s2s, x_species, x_species_phylo], axis=1)
    g = jax.nn.relu(_gat_layer(spin, params['sp0'], *pc_e, n_species,
                               True, True, **SCG))
    g = _gat_layer(g, params['sp1'], *pc_e, n_species, True, True, **SCG)
    g = jax.nn.relu(g)
    return _matmul(g, params['fc_W']) + params['fc_b']


# speedup vs baseline: 23.9781x; 1.1940x over previous
import functools

import jax
import jax.numpy as jnp
import numpy as np
from jax import lax
from jax.experimental import pallas as pl
from jax.experimental.pallas import tpu as pltpu
from jax.experimental.pallas import tpu_sc as plsc

HID = 64
AUGW = 80
AUXW = 16
LANES = 16
EC = 256


def _mm_body(x_ref, w_ref, o_ref):
    o_ref[...] = jnp.dot(x_ref[...], w_ref[...],
                         preferred_element_type=jnp.float32)


def _matmul(x, w, bm=400):
    n, k = x.shape
    _, m = w.shape
    return pl.pallas_call(
        _mm_body,
        grid=(n // bm,),
        in_specs=[pl.BlockSpec((bm, k), lambda i: (i, 0)),
                  pl.BlockSpec((k, m), lambda i: (0, 0))],
        out_specs=pl.BlockSpec((bm, m), lambda i: (i, 0)),
        out_shape=jax.ShapeDtypeStruct((n, m), jnp.float32),
    )(x, w)


def _augment_weights(p):
    W = p['W']
    din = W.shape[0]
    cols = [W, (W @ p['att_src'])[:, None], (W @ p['att_dst'])[:, None],
            jnp.zeros((din, AUGW - HID - 2), jnp.float32)]
    return jnp.concatenate(cols, axis=1)


SP_NLOC = 12544
SP_EPAD = 802816
CAP = 25344
NBINS_TOT = 32 * 4 * CAP
SB = 2 * EC


@functools.lru_cache(maxsize=None)
def _make_bin_kernel():
    per = SP_EPAD // 32
    nch = per // EC
    mesh = plsc.VectorSubcoreMesh(core_axis_name="c", subcore_axis_name="s")

    @functools.partial(
        pl.kernel, mesh=mesh,
        compiler_params=pltpu.CompilerParams(needs_layout_passes=False,
                                             use_tc_tiling_on_sc=False),
        out_type=(jax.ShapeDtypeStruct((NBINS_TOT,), jnp.int32),
                  jax.ShapeDtypeStruct((NBINS_TOT,), jnp.int32),
                  jax.ShapeDtypeStruct((NBINS_TOT,), jnp.float32),
                  jax.ShapeDtypeStruct((32 * 16,), jnp.int32)),
        scratch_types=(
            [pltpu.VMEM((EC,), jnp.int32),
             pltpu.VMEM((EC,), jnp.int32),
             pltpu.VMEM((EC,), jnp.float32)]
            + [pltpu.VMEM((SB,), jnp.int32) for _ in range(4)]
            + [pltpu.VMEM((SB,), jnp.int32) for _ in range(4)]
            + [pltpu.VMEM((SB,), jnp.float32) for _ in range(4)]
            + [pltpu.VMEM((16,), jnp.int32),
               pltpu.SemaphoreType.DMA]
        ),
    )
    def bk(srcp, dstp, eap, bsrc, bdst, bea, bcnt,
           src_v, dst_v, ea_v,
           ss0, ss1, ss2, ss3, ds0, ds1, ds2, ds3, es0, es1, es2, es3,
           cnt_v, sem):
        c = lax.axis_index("c")
        s = lax.axis_index("s")
        t = c * 16 + s
        ar = lax.iota(jnp.int32, LANES)
        zi = jnp.zeros((LANES,), jnp.int32)
        zf = jnp.zeros((LANES,), jnp.float32)
        stages = [(ss0, ds0, es0), (ss1, ds1, es1),
                  (ss2, ds2, es2), (ss3, ds3, es3)]

        def _chunk(g, carry):
            fills, offs = carry[:4], carry[4:]
            eb = pl.multiple_of(t * per + g * EC, 8)
            pltpu.sync_copy(srcp.at[pl.ds(eb, EC)], src_v)
            pltpu.sync_copy(dstp.at[pl.ds(eb, EC)], dst_v)
            pltpu.sync_copy(eap.at[pl.ds(eb, EC)], ea_v)

            def _grp(j, fills2):
                jo = j * LANES
                sv = src_v[pl.ds(jo, LANES)]
                d = dst_v[pl.ds(jo, LANES)]
                ev = ea_v[pl.ds(jo, LANES)]
                val = d >= 0
                bid = ((d >= SP_NLOC).astype(jnp.int32)
                       + (d >= 2 * SP_NLOC).astype(jnp.int32)
                       + (d >= 3 * SP_NLOC).astype(jnp.int32))
                new_fills = []
                for b in range(4):
                    mb = val & (bid == b)
                    mi = mb.astype(jnp.int32)
                    pos = fills2[b] + plsc.cumsum(mi) - 1
                    sb, db, eab = stages[b]
                    plsc.store_scatter(sb, [pos], sv, mask=mb)
                    plsc.store_scatter(db, [pos], d, mask=mb)
                    plsc.store_scatter(eab, [pos], ev, mask=mb)
                    new_fills.append(fills2[b] + jnp.sum(mi))
                return tuple(new_fills)
            fills = lax.fori_loop(0, EC // LANES, _grp, tuple(fills))

            new_carry = []
            for b in range(4):
                sb, db, eab = stages[b]
                do_flush = fills[b] >= EC
                hoff = pl.multiple_of((t * 4 + b) * CAP + offs[b], 8)

                @pl.when(do_flush)
                def _flush(sb=sb, db=db, eab=eab, hoff=hoff):
                    pltpu.sync_copy(sb.at[pl.ds(0, EC)],
                                    bsrc.at[pl.ds(hoff, EC)])
                    pltpu.sync_copy(db.at[pl.ds(0, EC)],
                                    bdst.at[pl.ds(hoff, EC)])
                    pltpu.sync_copy(eab.at[pl.ds(0, EC)],
                                    bea.at[pl.ds(hoff, EC)])
                    for k in range(EC // LANES):
                        o = pl.ds(k * LANES, LANES)
                        o2 = pl.ds(EC + k * LANES, LANES)
                        sb[o] = sb[o2]
                        db[o] = db[o2]
                        eab[o] = eab[o2]
                new_carry.append(jnp.where(do_flush, fills[b] - EC,
                                           fills[b]))
            for b in range(4):
                new_carry.append(offs[b]
                                 + jnp.where(fills[b] >= EC, EC, 0))
            return tuple(new_carry)
        z = jnp.int32(0)
        carry = lax.fori_loop(0, nch, _chunk, (z,) * 8)
        fills, offs = carry[:4], carry[4:]

        cvec = zi
        for b in range(4):
            sb, db, eab = stages[b]
            fill = fills[b]
            for k in range(EC // LANES):
                pos = k * LANES + ar
                mz = pos >= fill
                plsc.store_scatter(sb, [pos], zi, mask=mz)
                plsc.store_scatter(db, [pos], zi, mask=mz)
                plsc.store_scatter(eab, [pos], zf, mask=mz)
            hoff = pl.multiple_of((t * 4 + b) * CAP + offs[b], 8)
            pltpu.sync_copy(sb.at[pl.ds(0, EC)], bsrc.at[pl.ds(hoff, EC)])
            pltpu.sync_copy(db.at[pl.ds(0, EC)], bdst.at[pl.ds(hoff, EC)])
            pltpu.sync_copy(eab.at[pl.ds(0, EC)], bea.at[pl.ds(hoff, EC)])
            cvec = jnp.where(ar == b,
                             jnp.broadcast_to(offs[b] + fill, (LANES,)),
                             cvec)
        cnt_v[pl.ds(0, LANES)] = cvec
        pltpu.sync_copy(cnt_v, bcnt.at[pl.ds(pl.multiple_of(t * 16, 8), 16)])

    return bk


@functools.lru_cache(maxsize=None)
def _make_binned_layer_kernel(step, use_dst):
    nloc = SP_NLOC
    zc = 196
    nrows = nloc // 16
    nz = nrows // zc
    mesh = plsc.VectorSubcoreMesh(core_axis_name="c", subcore_axis_name="s")

    @functools.partial(
        pl.kernel, mesh=mesh,
        compiler_params=pltpu.CompilerParams(needs_layout_passes=False,
                                             use_tc_tiling_on_sc=False),
        out_type=(jax.ShapeDtypeStruct((2, nloc, HID), jnp.float32),
                  jax.ShapeDtypeStruct((2, nloc, AUXW), jnp.float32)),
        scratch_types=[
            pltpu.VMEM((EC,), jnp.int32),
            pltpu.VMEM((EC,), jnp.int32),
            pltpu.VMEM((EC,), jnp.float32),
            pltpu.VMEM((EC,), jnp.int32),
            pltpu.VMEM((EC,), jnp.float32),
            pltpu.VMEM((EC,), jnp.float32),
            pltpu.VMEM((EC, HID), jnp.float32),
            pltpu.VMEM((EC, AUXW), jnp.float32),
            pltpu.VMEM((32 * 16,), jnp.int32),
            pltpu.VMEM((16,), jnp.float32),
            pltpu.VMEM_SHARED((nloc, HID), jnp.float32),
            pltpu.VMEM_SHARED((nloc, AUXW), jnp.float32),
            pltpu.SemaphoreType.DMA,
        ],
    )
    def ek(h64, bsrc, bdst, bea, bcnt, asrcp, adstp, cvecp, out64, outA,
           src_v, dst_v, ea_v, dstl_v, asrc_c, adst_c,
           rows_v, aux_v, cnt_v, cv_v, acc64, accA, sem):
        c = lax.axis_index("c")
        s = lax.axis_index("s")
        ar = lax.iota(jnp.int32, LANES)
        zf = jnp.zeros((LANES,), jnp.float32)
        is0, is1, is2 = (ar == 0), (ar == 1), (ar == 2)
        q = 2 * step + c
        dst_lo = q * nloc

        def _zr(r, carry):
            for k in range(HID // LANES):
                rows_v[r, pl.ds(k * LANES, LANES)] = zf
            aux_v[r, pl.ds(0, LANES)] = zf
            return carry
        lax.fori_loop(0, zc, _zr, 0)
        r0 = s * nrows
        for z in range(nz):
            pltpu.sync_copy(rows_v.at[pl.ds(0, zc)],
                            acc64.at[pl.ds(r0 + z * zc, zc)])
            pltpu.sync_copy(aux_v.at[pl.ds(0, zc)],
                            accA.at[pl.ds(r0 + z * zc, zc)])
        pltpu.sync_copy(bcnt, cnt_v)
        pltpu.sync_copy(cvecp, cv_v)
        def _za(r, carry):
            aux_v[r, pl.ds(0, LANES)] = zf
            return carry
        lax.fori_loop(0, EC, _za, 0)
        plsc.subcore_barrier()
        cvec = cv_v[pl.ds(0, LANES)]

        for bi in range(2):
            t = 2 * s + bi
            cnt16 = cnt_v[pl.ds(t * 16, LANES)]
            nb = cnt16.at[ar - ar + q].get(mode='promise_in_bounds')[0]
            nch = (nb + EC - 1) // EC
            bin_base = (t * 4 + q) * CAP

            def _chunk(g, carry):
                eb = pl.multiple_of(bin_base + g * EC, 8)
                d1 = pltpu.async_copy(bsrc.at[pl.ds(eb, EC)], src_v, sem)
                d2 = pltpu.async_copy(bdst.at[pl.ds(eb, EC)], dst_v, sem)
                d3 = pltpu.async_copy(bea.at[pl.ds(eb, EC)], ea_v, sem)
                d1.wait()
                d2.wait()
                d3.wait()
                g1 = pltpu.async_copy(h64.at[src_v], rows_v, sem)
                g2 = pltpu.async_copy(asrcp.at[src_v], asrc_c, sem)
                if use_dst:
                    g3 = pltpu.async_copy(adstp.at[dst_v], adst_c, sem)
                g1.wait()
                g2.wait()
                if use_dst:
                    g3.wait()

                def _grpB(j, carry2):
                    jo = j * LANES
                    dst16 = dst_v[pl.ds(jo, LANES)]
                    ea16 = ea_v[pl.ds(jo, LANES)]
                    m = (g * EC + jo + ar) < nb
                    dstl = dst16 - dst_lo
                    dstl_v[pl.ds(jo, LANES)] = jnp.where(m, dstl, 0)
                    a = asrc_c[pl.ds(jo, LANES)] + ea16 * cvec
                    if use_dst:
                        a = a + adst_c[pl.ds(jo, LANES)]
                    a = jnp.where(a >= 0.0, a, 0.2 * a)
                    ex = jnp.where(m, jnp.exp(a), 0.0)
                    one = jnp.where(m, 1.0, 0.0)
                    eam = jnp.where(m, ea16 * cvec, 0.0)
                    arow = jo + ar
                    plsc.store_scatter(aux_v, [arow, ar - ar], ex)
                    plsc.store_scatter(aux_v, [arow, ar - ar + 1], eam)
                    plsc.store_scatter(aux_v, [arow, ar - ar + 2], one)
                    for jj in range(LANES):
                        sel = ar - ar + jj
                        bex = ex.at[sel].get(mode='promise_in_bounds')
                        for k in range(HID // LANES):
                            sl = pl.ds(k * LANES, LANES)
                            rows_v[jo + jj, sl] = rows_v[jo + jj, sl] * bex
                    return carry2
                lax.fori_loop(0, EC // LANES, _grpB, 0)
                pltpu.sync_copy(rows_v, acc64.at[dstl_v], add=True)
                pltpu.sync_copy(aux_v, accA.at[dstl_v], add=True)
                return carry
            lax.fori_loop(0, nch, _chunk, 0)
        plsc.subcore_barrier()
        pltpu.sync_copy(acc64.at[pl.ds(r0, nrows)],
                        out64.at[c, pl.ds(r0, nrows)])
        pltpu.sync_copy(accA.at[pl.ds(r0, nrows)],
                        outA.at[c, pl.ds(r0, nrows)])

    return ek


@functools.lru_cache(maxsize=None)
def _make_edge_kernel(e_pad, nloc, zc, split, use_dst, step):
    nrows = nloc // 16
    nz = nrows // zc
    assert nrows % zc == 0 and zc <= EC and nloc % 16 == 0
    per = e_pad // 16 if split else e_pad // 32
    nch = per // EC
    assert per % EC == 0

    mesh = plsc.VectorSubcoreMesh(core_axis_name="c", subcore_axis_name="s")

    @functools.partial(
        pl.kernel, mesh=mesh,
        compiler_params=pltpu.CompilerParams(needs_layout_passes=False,
                                             use_tc_tiling_on_sc=False),
        out_type=(jax.ShapeDtypeStruct((2, nloc, HID), jnp.float32),
                  jax.ShapeDtypeStruct((2, nloc, AUXW), jnp.float32)),
        scratch_types=[
            pltpu.VMEM((EC,), jnp.int32),
            pltpu.VMEM((EC,), jnp.int32),
            pltpu.VMEM((EC,), jnp.float32),
            pltpu.VMEM((EC,), jnp.int32),
            pltpu.VMEM((EC,), jnp.int32),
            pltpu.VMEM((EC,), jnp.float32),
            pltpu.VMEM((EC,), jnp.float32),
            pltpu.VMEM((EC, HID), jnp.float32),
            pltpu.VMEM((EC, AUXW), jnp.float32),
            pltpu.VMEM_SHARED((nloc, HID), jnp.float32),
            pltpu.VMEM_SHARED((nloc, AUXW), jnp.float32),
            pltpu.SemaphoreType.DMA,
        ],
    )
    def ek(h64, srcp, dstp, eap, asrcp, adstp, out64, outA,
           src_v, dst_v, ea_v, dstl_v, dstg_v, asrc_c, adst_c,
           rows_v, aux_v, acc64, accA, sem):
        c = lax.axis_index("c")
        s = lax.axis_index("s")
        ar = lax.iota(jnp.int32, LANES)
        zf = jnp.zeros((LANES,), jnp.float32)
        is0, is1, is2 = (ar == 0), (ar == 1), (ar == 2)

        def _zr(r, carry):
            for k in range(HID // LANES):
                rows_v[r, pl.ds(k * LANES, LANES)] = zf
            aux_v[r, pl.ds(0, LANES)] = zf
            return carry
        lax.fori_loop(0, zc, _zr, 0)
        r0 = s * nrows
        for z in range(nz):
            pltpu.sync_copy(rows_v.at[pl.ds(0, zc)],
                            acc64.at[pl.ds(r0 + z * zc, zc)])
            pltpu.sync_copy(aux_v.at[pl.ds(0, zc)],
                            accA.at[pl.ds(r0 + z * zc, zc)])
        plsc.subcore_barrier()

        dst_lo = (2 * step + c) * nloc if split else 0
        base = s * per if split else (s * 2 + c) * per

        def _chunk(g, carry):
            eb = base + g * EC
            pltpu.sync_copy(srcp.at[pl.ds(eb, EC)], src_v)
            pltpu.sync_copy(dstp.at[pl.ds(eb, EC)], dst_v)
            pltpu.sync_copy(eap.at[pl.ds(eb, EC)], ea_v)
            pltpu.async_copy(h64.at[src_v], rows_v, sem).wait()
            pltpu.async_copy(asrcp.at[src_v], asrc_c, sem).wait()

            def _grpA(j, carry2):
                jo = j * LANES
                dst16 = dst_v[pl.ds(jo, LANES)]
                dstl = dst16 - dst_lo
                m = (dstl >= 0) & (dstl < nloc)
                dstl_v[pl.ds(jo, LANES)] = jnp.where(m, dstl, 0)
                dstg_v[pl.ds(jo, LANES)] = jnp.where(m, dst16, 0)
                return carry2
            lax.fori_loop(0, EC // LANES, _grpA, 0)
            if use_dst:
                pltpu.async_copy(adstp.at[dstg_v], adst_c, sem).wait()

            def _grpB(j, carry2):
                jo = j * LANES
                dstl = dstl_v[pl.ds(jo, LANES)]
                dst16 = dst_v[pl.ds(jo, LANES)]
                ea16 = ea_v[pl.ds(jo, LANES)]
                m = (dst16 - dst_lo >= 0) & (dst16 - dst_lo < nloc)
                a = asrc_c[pl.ds(jo, LANES)] + ea16
                if use_dst:
                    a = a + adst_c[pl.ds(jo, LANES)]
                a = jnp.where(a >= 0.0, a, 0.2 * a)
                ex = jnp.where(m, jnp.exp(a), 0.0)
                one = jnp.where(m, 1.0, 0.0)
                eam = jnp.where(m, ea16, 0.0)
                for jj in range(LANES):
                    sel = ar - ar + jj
                    bex = ex.at[sel].get(mode='promise_in_bounds')
                    bea = eam.at[sel].get(mode='promise_in_bounds')
                    bone = one.at[sel].get(mode='promise_in_bounds')
                    aux = jnp.where(is0, bex,
                                    jnp.where(is1, bea,
                                              jnp.where(is2, bone, zf)))
                    aux_v[jo + jj, pl.ds(0, LANES)] = aux
                    for k in range(HID // LANES):
                        sl = pl.ds(k * LANES, LANES)
                        rows_v[jo + jj, sl] = rows_v[jo + jj, sl] * bex
                return carry2
            lax.fori_loop(0, EC // LANES, _grpB, 0)
            pltpu.sync_copy(rows_v, acc64.at[dstl_v], add=True)
            pltpu.sync_copy(aux_v, accA.at[dstl_v], add=True)
            return carry
        lax.fori_loop(0, nch, _chunk, 0)
        plsc.subcore_barrier()
        pltpu.sync_copy(acc64.at[pl.ds(r0, nrows)],
                        out64.at[c, pl.ds(r0, nrows)])
        pltpu.sync_copy(accA.at[pl.ds(r0, nrows)],
                        outA.at[c, pl.ds(r0, nrows)])

    return ek


def _pad1(a, n, fill):
    return jnp.pad(a, (0, n - a.shape[0]), constant_values=fill)


def _edge_pass_sc(h64, src_p, dst_p, ea_p, asrc_arr, adst_arr, num_dst,
                  nloc, zc, split, use_dst):
    n_steps = 2 if split else 1
    p64, pA = [], []
    for step in range(n_steps):
        ek = _make_edge_kernel(src_p.shape[0], nloc, zc, split, use_dst,
                               step)
        o64, oA = ek(h64, src_p, dst_p, ea_p, asrc_arr, adst_arr)
        p64 += [o64[0], o64[1]]
        pA += [oA[0], oA[1]]
    if split:
        return (jnp.concatenate(p64, axis=0)[:num_dst],
                jnp.concatenate(pA, axis=0)[:num_dst])
    return (p64[0] + p64[1])[:num_dst], (pA[0] + pA[1])[:num_dst]


def _edge_pass_binned(h64, bins, asrc_arr, adst_arr, cval, num_dst, use_dst):
    bsrc, bdst, bea, bcnt = bins
    cvec = jnp.full((16,), cval, jnp.float32)
    p64, pA = [], []
    for step in range(2):
        ek = _make_binned_layer_kernel(step, use_dst)
        o64, oA = ek(h64, bsrc, bdst, bea, bcnt, asrc_arr, adst_arr, cvec)
        p64 += [o64[0], o64[1]]
        pA += [oA[0], oA[1]]
    return (jnp.concatenate(p64, axis=0)[:num_dst],
            jnp.concatenate(pA, axis=0)[:num_dst])


def _gat_layer(x, p, src_p, dst_p, ea_raw_p, num_dst, self_loops, use_dst,
               nloc, zc, split, bins=None):
    c = jnp.dot(p['W_e'][0], p['att_edge'])
    ea_p = c * ea_raw_p
    Waug = _augment_weights(p)
    H = _matmul(x, Waug)
    h64 = H[:, :HID]
    asrc_arr = H[:, HID]
    adst_arr = H[:, HID + 1] if use_dst else jnp.zeros((16,), jnp.float32)
    if bins is not None:
        num, stats = _edge_pass_binned(h64, bins, asrc_arr, adst_arr, c,
                                       num_dst, use_dst)
    else:
        num, stats = _edge_pass_sc(h64, src_p, dst_p, ea_p, asrc_arr,
                                   adst_arr, num_dst, nloc, zc, split,
                                   use_dst)
    exsum, easum, cnt = stats[:, 0], stats[:, 1], stats[:, 2]
    if self_loops:
        loop_ea = easum / jnp.maximum(cnt, 1.0)
        a_loop = asrc_arr + adst_arr + loop_ea
        a_loop = jnp.where(a_loop >= 0, a_loop, 0.2 * a_loop)
        ex_loop = jnp.exp(a_loop)
        num = num + ex_loop[:, None] * h64
        denom = exsum + ex_loop + 1e-16
    else:
        denom = exsum + 1e-16
    return num / denom[:, None] + p['b']


def kernel(x_spatial, global_data, edge_index_spatial, edge_attr_spatial,
           bip_edge_index, bip_edge_attr, x_species, x_species_phylo,
           edge_index_species, edge_attr_species, params):
    n_space = x_spatial.shape[0]
    n_species = x_species.shape[0]

    def _prep(src, dst, ea, e_pad):
        return (_pad1(src, e_pad, 0), _pad1(dst, e_pad, -1),
                _pad1(ea[:, 0], e_pad, 0.0))

    sp_e = _prep(edge_index_spatial[0], edge_index_spatial[1],
                 edge_attr_spatial, SP_EPAD)
    bp_e = _prep(bip_edge_index[0], bip_edge_index[1],
                 bip_edge_attr, 507904)
    pc_e = _prep(edge_index_species[0], edge_index_species[1],
                 edge_attr_species, 163840)

    SPG = dict(nloc=SP_NLOC, zc=196, split=True)
    SCG = dict(nloc=10000, zc=125, split=False)

    bins = _make_bin_kernel()(*sp_e)

    si = jnp.concatenate([x_spatial, global_data], axis=1)
    h = jax.nn.relu(_gat_layer(si, params['space0'], *sp_e, n_space,
                               True, True, bins=bins, **SPG))
    h = _gat_layer(h, params['space1'], *sp_e, n_space, True, True,
                   bins=bins, **SPG)
    h = jax.nn.relu(h)
    s2s = _gat_layer(h, params['bip'], *bp_e, n_species, False, False, **SCG)
    spin = jnp.concatenate([s2s, x_species, x_species_phylo], axis=1)
    g = jax.nn.relu(_gat_layer(spin, params['sp0'], *pc_e, n_species,
                               True, True, **SCG))
    g = _gat_layer(g, params['sp1'], *pc_e, n_species, True, True, **SCG)
    g = jax.nn.relu(g)
    return _matmul(g, params['fc_W']) + params['fc_b']
